# baseline (speedup 1.0000x reference)
import functools

import jax
import jax.numpy as jnp
from jax import lax
from jax.experimental import pallas as pl
from jax.experimental.pallas import tpu as pltpu
from jax.experimental.pallas import tpu_sc as plsc

N = 10000
E = 320000
F = 128
H = 128
G = 64

NS = 16
NW = 2 * NS
EPT = E // NW
K = 50
NCHUNK = EPT // K
NSTAGE = 5
STAGE = NCHUNK // NSTAGE
RPT = 624
TAIL = N - NS * RPT
BLK = 1000
NBLK = N // BLK


def _sc_agg_body(h, src4, dst4, out, src_v, dst_v, rows_0, rows_1, rows_2,
                 rows_3, gsem_a, gsem_b, ssem_a, ssem_b, acc_s):
    c = lax.axis_index("c")
    s = lax.axis_index("s")
    wid = c * NS + s
    r0 = s * RPT
    last = s == NS - 1
    t0 = NS * RPT

    z16 = jnp.zeros((16,), jnp.float32)

    def zfill(k, _):
        rows_0[k // 8, pl.ds((k % 8) * 16, 16)] = z16
        rows_1[k // 8, pl.ds((k % 8) * 16, 16)] = z16
        return 0

    lax.fori_loop(0, (K * F) // 16, zfill, 0)
    for t in range(RPT // (2 * K)):
        pltpu.sync_copy(rows_0, acc_s.at[pl.ds(r0 + 2 * t * K, K)])
        pltpu.sync_copy(rows_1, acc_s.at[pl.ds(r0 + (2 * t + 1) * K, K)])
    rem = RPT - (RPT // (2 * K)) * 2 * K
    pltpu.sync_copy(rows_0.at[pl.ds(0, rem)],
                    acc_s.at[pl.ds(r0 + RPT - rem, rem)])

    @pl.when(last)
    def _():
        pltpu.sync_copy(rows_1.at[pl.ds(0, TAIL)], acc_s.at[pl.ds(t0, TAIL)])

    plsc.subcore_barrier()

    def gath(i, buf, sem):
        pltpu.async_copy(h.at[src_v.at[i]], buf, sem)

    def gath_w(i, buf, sem):
        pltpu.make_async_copy(h.at[src_v.at[i]], buf, sem).wait()

    def scat(i, buf, sem):
        pltpu.async_copy(buf, acc_s.at[dst_v.at[i]], sem, add=True)

    def scat_w(i, buf, sem):
        pltpu.make_async_copy(buf, acc_s.at[dst_v.at[i]], sem).wait()

    for ss in range(NSTAGE):
        pltpu.sync_copy(src4.at[wid, ss], src_v)
        pltpu.sync_copy(dst4.at[wid, ss], dst_v)

        gath(0, rows_0, gsem_a)
        gath(1, rows_1, gsem_a)

        def body(j, _):
            i = j * 4
            gath_w(i, rows_0, gsem_a)
            gath_w(i + 1, rows_1, gsem_a)

            @pl.when(j > 0)
            def _():
                scat_w(i - 2, rows_2, ssem_b)
                scat_w(i - 1, rows_3, ssem_b)

            gath(i + 2, rows_2, gsem_b)
            gath(i + 3, rows_3, gsem_b)
            scat(i, rows_0, ssem_a)
            scat(i + 1, rows_1, ssem_a)

            gath_w(i + 2, rows_2, gsem_b)
            gath_w(i + 3, rows_3, gsem_b)
            scat_w(i, rows_0, ssem_a)
            scat_w(i + 1, rows_1, ssem_a)

            @pl.when(j < STAGE // 4 - 1)
            def _():
                gath(i + 4, rows_0, gsem_a)
                gath(i + 5, rows_1, gsem_a)

            scat(i + 2, rows_2, ssem_b)
            scat(i + 3, rows_3, ssem_b)
            return 0

        lax.fori_loop(0, STAGE // 4, body, 0)
        scat_w(STAGE - 2, rows_2, ssem_b)
        scat_w(STAGE - 1, rows_3, ssem_b)

    plsc.subcore_barrier()

    pltpu.sync_copy(acc_s.at[pl.ds(r0, RPT)], out.at[c, pl.ds(r0, RPT)])

    @pl.when(last)
    def _():
        pltpu.sync_copy(acc_s.at[pl.ds(t0, TAIL)], out.at[c, pl.ds(t0, TAIL)])


_sc_agg = pl.kernel(
    _sc_agg_body,
    mesh=plsc.VectorSubcoreMesh(core_axis_name="c", subcore_axis_name="s"),
    out_type=jax.ShapeDtypeStruct((2, N, F), jnp.float32),
    scratch_types=[
        pltpu.VMEM((STAGE, K), jnp.int32),
        pltpu.VMEM((STAGE, K), jnp.int32),
        pltpu.VMEM((K, F), jnp.float32),
        pltpu.VMEM((K, F), jnp.float32),
        pltpu.VMEM((K, F), jnp.float32),
        pltpu.VMEM((K, F), jnp.float32),
        pltpu.SemaphoreType.DMA,
        pltpu.SemaphoreType.DMA,
        pltpu.SemaphoreType.DMA,
        pltpu.SemaphoreType.DMA,
        pltpu.VMEM_SHARED((N, F), jnp.float32),
    ],
)


def _mlp1_body(x_ref, a_ref, w1_ref, b1_ref, w2_ref, b2_ref, o_ref):
    a = a_ref[...]
    z = x_ref[...] + a[0] + a[1]
    z = jnp.maximum(
        jnp.dot(z, w1_ref[...], preferred_element_type=jnp.float32)
        + b1_ref[...], 0.0)
    z = jnp.maximum(
        jnp.dot(z, w2_ref[...], preferred_element_type=jnp.float32)
        + b2_ref[...], 0.0)
    o_ref[...] = z


def _mlp1(h, agg, w1, b1, w2, b2):
    return pl.pallas_call(
        _mlp1_body,
        grid=(NBLK,),
        in_specs=[
            pl.BlockSpec((BLK, F), lambda i: (i, 0)),
            pl.BlockSpec((2, BLK, F), lambda i: (0, i, 0)),
            pl.BlockSpec((F, H), lambda i: (0, 0)),
            pl.BlockSpec((1, H), lambda i: (0, 0)),
            pl.BlockSpec((H, H), lambda i: (0, 0)),
            pl.BlockSpec((1, H), lambda i: (0, 0)),
        ],
        out_specs=pl.BlockSpec((BLK, H), lambda i: (i, 0)),
        out_shape=jax.ShapeDtypeStruct((N, H), jnp.float32),
        compiler_params=pltpu.CompilerParams(
            dimension_semantics=("arbitrary",)),
    )(h, agg, w1, b1, w2, b2)


def _mlp2_body(h_ref, a_ref, batch_ref, w1_ref, b1_ref, w2_ref, b2_ref,
               l1w_ref, l1b_ref, l2w_ref, l2b_ref, outw_ref, outb_ref,
               o_ref, sums, counts):
    i = pl.program_id(0)

    @pl.when(i == 0)
    def _():
        sums[...] = jnp.zeros((G, H), jnp.float32)
        counts[...] = jnp.zeros((G, H), jnp.float32)

    a = a_ref[...]
    z = h_ref[...] + a[0] + a[1]
    z = jnp.maximum(
        jnp.dot(z, w1_ref[...], preferred_element_type=jnp.float32)
        + b1_ref[...], 0.0)
    h2 = jnp.maximum(
        jnp.dot(z, w2_ref[...], preferred_element_type=jnp.float32)
        + b2_ref[...], 0.0)

    bt = batch_ref[0, 0, :]
    oh = (bt[:, None] == lax.broadcasted_iota(jnp.int32, (BLK, G), 1))
    oh = oh.astype(jnp.float32)
    sums[...] += lax.dot_general(
        oh, h2, dimension_numbers=(((0,), (0,)), ((), ())),
        preferred_element_type=jnp.float32)
    counts[...] += jnp.broadcast_to(jnp.sum(oh, axis=0)[:, None], (G, H))

    @pl.when(i == NBLK - 1)
    def _():
        pooled = sums[...] / jnp.maximum(counts[...], 1.0)
        y = jnp.maximum(
            jnp.dot(pooled, l1w_ref[...], preferred_element_type=jnp.float32)
            + l1b_ref[...], 0.0)
        y = jnp.maximum(
            jnp.dot(y, l2w_ref[...], preferred_element_type=jnp.float32)
            + l2b_ref[...], 0.0)
        yv = jnp.sum(y * outw_ref[...], axis=1)[:, None] + outb_ref[...]
        o_ref[...] = jnp.broadcast_to(yv, (G, H))


def _mlp2(h, agg, batch3, w1, b1, w2, b2, l1w, l1b, l2w, l2b, outw, outb):
    return pl.pallas_call(
        _mlp2_body,
        grid=(NBLK,),
        in_specs=[
            pl.BlockSpec((BLK, F), lambda i: (i, 0)),
            pl.BlockSpec((2, BLK, F), lambda i: (0, i, 0)),
            pl.BlockSpec((1, 1, BLK), lambda i: (i, 0, 0)),
            pl.BlockSpec((H, H), lambda i: (0, 0)),
            pl.BlockSpec((1, H), lambda i: (0, 0)),
            pl.BlockSpec((H, H), lambda i: (0, 0)),
            pl.BlockSpec((1, H), lambda i: (0, 0)),
            pl.BlockSpec((H, H), lambda i: (0, 0)),
            pl.BlockSpec((1, H), lambda i: (0, 0)),
            pl.BlockSpec((H, H), lambda i: (0, 0)),
            pl.BlockSpec((1, H), lambda i: (0, 0)),
            pl.BlockSpec((1, H), lambda i: (0, 0)),
            pl.BlockSpec((1, 1), lambda i: (0, 0)),
        ],
        out_specs=pl.BlockSpec((G, H), lambda i: (0, 0)),
        out_shape=jax.ShapeDtypeStruct((G, H), jnp.float32),
        scratch_shapes=[
            pltpu.VMEM((G, H), jnp.float32),
            pltpu.VMEM((G, H), jnp.float32),
        ],
        compiler_params=pltpu.CompilerParams(
            dimension_semantics=("arbitrary",)),
    )(h, agg, batch3, w1, b1, w2, b2, l1w, l1b, l2w, l2b, outw, outb)


def kernel(x, edge_index, batch, g1_W1, g1_b1, g1_W2, g1_b2,
           g2_W1, g2_b1, g2_W2, g2_b2, l1_W, l1_b, l2_W, l2_b,
           out_W, out_b):
    src3 = edge_index[0].reshape(NW, NSTAGE, STAGE, K)
    dst3 = edge_index[1].reshape(NW, NSTAGE, STAGE, K)
    batch3 = batch.reshape(NBLK, 1, BLK)

    a1 = _sc_agg(x, src3, dst3)
    h1 = _mlp1(x, a1, g1_W1, g1_b1.reshape(1, H), g1_W2, g1_b2.reshape(1, H))
    a2 = _sc_agg(h1, src3, dst3)
    yf = _mlp2(h1, a2, batch3, g2_W1, g2_b1.reshape(1, H), g2_W2,
               g2_b2.reshape(1, H), l1_W, l1_b.reshape(1, H), l2_W,
               l2_b.reshape(1, H), out_W.reshape(1, H), out_b.reshape(1, 1))
    return yf[:, 0]


# speedup vs baseline: 1.0763x; 1.0763x over previous
import functools

import jax
import jax.numpy as jnp
from jax import lax
from jax.experimental import pallas as pl
from jax.experimental.pallas import tpu as pltpu
from jax.experimental.pallas import tpu_sc as plsc

N = 10000
E = 320000
F = 128
H = 128
G = 64

NS = 16
NW = 2 * NS
EPT = E // NW
K = 50
NCHUNK = EPT // K
NSTAGE = 2
STAGE = NCHUNK // NSTAGE
RPT = 624
TAIL = N - NS * RPT
BLK = 2000
NBLK = N // BLK


def _sc_agg_body(h, src4, dst4, out, src_v, dst_v, rows_a, rows_b,
                 gsem_a, gsem_b, zsem, acc_s):
    c = lax.axis_index("c")
    s = lax.axis_index("s")
    wid = c * NS + s
    r0 = s * RPT
    last = s == NS - 1
    t0 = NS * RPT

    pltpu.async_copy(src4.at[wid, 0], src_v, zsem)
    pltpu.async_copy(dst4.at[wid, 0], dst_v, zsem)

    z16 = jnp.zeros((16,), jnp.float32)

    def zfill(k, _):
        rows_a[k // 8, pl.ds((k % 8) * 16, 16)] = z16
        rows_b[k // 8, pl.ds((k % 8) * 16, 16)] = z16
        return 0

    lax.fori_loop(0, (K * F) // 16, zfill, 0)
    pltpu.make_async_copy(src4.at[wid, 0], src_v, zsem).wait()
    pltpu.make_async_copy(dst4.at[wid, 0], dst_v, zsem).wait()
    nz = RPT // (2 * K)
    for t in range(nz):
        pltpu.async_copy(rows_a, acc_s.at[pl.ds(r0 + 2 * t * K, K)], zsem)
        pltpu.async_copy(rows_b, acc_s.at[pl.ds(r0 + (2 * t + 1) * K, K)],
                         zsem)
    rem = RPT - nz * 2 * K
    pltpu.async_copy(rows_a.at[pl.ds(0, rem)],
                     acc_s.at[pl.ds(r0 + RPT - rem, rem)], zsem)

    @pl.when(last)
    def _():
        pltpu.sync_copy(rows_b.at[pl.ds(0, TAIL)], acc_s.at[pl.ds(t0, TAIL)])

    for t in range(nz):
        pltpu.make_async_copy(rows_a, acc_s.at[pl.ds(r0, K)], zsem).wait()
        pltpu.make_async_copy(rows_b, acc_s.at[pl.ds(r0, K)], zsem).wait()
    pltpu.make_async_copy(rows_a.at[pl.ds(0, rem)],
                          acc_s.at[pl.ds(r0, rem)], zsem).wait()

    pltpu.async_copy(h.at[src_v.at[0]], rows_a, gsem_a)

    plsc.subcore_barrier()

    for hh in range(NSTAGE):
        if hh > 0:
            pltpu.sync_copy(src4.at[wid, hh], src_v)
            pltpu.sync_copy(dst4.at[wid, hh], dst_v)
            pltpu.async_copy(h.at[src_v.at[0]], rows_a, gsem_a)

        def body(j, _):
            i = j * 2
            pltpu.async_copy(h.at[src_v.at[i + 1]], rows_b, gsem_b)
            pltpu.make_async_copy(h.at[src_v.at[i]], rows_a, gsem_a).wait()
            pltpu.sync_copy(rows_a, acc_s.at[dst_v.at[i]], add=True)

            @pl.when(j < STAGE // 2 - 1)
            def _():
                pltpu.async_copy(h.at[src_v.at[i + 2]], rows_a, gsem_a)

            pltpu.make_async_copy(h.at[src_v.at[i + 1]], rows_b, gsem_b).wait()
            pltpu.sync_copy(rows_b, acc_s.at[dst_v.at[i + 1]], add=True)
            return 0

        lax.fori_loop(0, STAGE // 2, body, 0)

    plsc.subcore_barrier()

    pltpu.sync_copy(acc_s.at[pl.ds(r0, RPT)], out.at[c, pl.ds(r0, RPT)])

    @pl.when(last)
    def _():
        pltpu.sync_copy(acc_s.at[pl.ds(t0, TAIL)], out.at[c, pl.ds(t0, TAIL)])


_sc_agg = pl.kernel(
    _sc_agg_body,
    mesh=plsc.VectorSubcoreMesh(core_axis_name="c", subcore_axis_name="s"),
    out_type=jax.ShapeDtypeStruct((2, N, F), jnp.float32),
    scratch_types=[
        pltpu.VMEM((STAGE, K), jnp.int32),
        pltpu.VMEM((STAGE, K), jnp.int32),
        pltpu.VMEM((K, F), jnp.float32),
        pltpu.VMEM((K, F), jnp.float32),
        pltpu.SemaphoreType.DMA,
        pltpu.SemaphoreType.DMA,
        pltpu.SemaphoreType.DMA,
        pltpu.VMEM_SHARED((N, F), jnp.float32),
    ],
)


def _mlp1_body(x_ref, a_ref, w1_ref, b1_ref, w2_ref, b2_ref, o_ref):
    a = a_ref[...]
    z = x_ref[...] + a[0] + a[1]
    z = jnp.maximum(
        jnp.dot(z, w1_ref[...], preferred_element_type=jnp.float32)
        + b1_ref[...], 0.0)
    z = jnp.maximum(
        jnp.dot(z, w2_ref[...], preferred_element_type=jnp.float32)
        + b2_ref[...], 0.0)
    o_ref[...] = z


def _mlp1(h, agg, w1, b1, w2, b2):
    return pl.pallas_call(
        _mlp1_body,
        grid=(NBLK,),
        in_specs=[
            pl.BlockSpec((BLK, F), lambda i: (i, 0)),
            pl.BlockSpec((2, BLK, F), lambda i: (0, i, 0)),
            pl.BlockSpec((F, H), lambda i: (0, 0)),
            pl.BlockSpec((1, H), lambda i: (0, 0)),
            pl.BlockSpec((H, H), lambda i: (0, 0)),
            pl.BlockSpec((1, H), lambda i: (0, 0)),
        ],
        out_specs=pl.BlockSpec((BLK, H), lambda i: (i, 0)),
        out_shape=jax.ShapeDtypeStruct((N, H), jnp.float32),
        compiler_params=pltpu.CompilerParams(
            dimension_semantics=("arbitrary",)),
    )(h, agg, w1, b1, w2, b2)


def _mlp2_body(h_ref, a_ref, batch_ref, w1_ref, b1_ref, w2_ref, b2_ref,
               l1w_ref, l1b_ref, l2w_ref, l2b_ref, outw_ref, outb_ref,
               o_ref, sums, counts):
    i = pl.program_id(0)

    @pl.when(i == 0)
    def _():
        sums[...] = jnp.zeros((G, H), jnp.float32)
        counts[...] = jnp.zeros((G, H), jnp.float32)

    a = a_ref[...]
    z = h_ref[...] + a[0] + a[1]
    z = jnp.maximum(
        jnp.dot(z, w1_ref[...], preferred_element_type=jnp.float32)
        + b1_ref[...], 0.0)
    h2 = jnp.maximum(
        jnp.dot(z, w2_ref[...], preferred_element_type=jnp.float32)
        + b2_ref[...], 0.0)

    bt = batch_ref[0, 0, :]
    oh = (bt[:, None] == lax.broadcasted_iota(jnp.int32, (BLK, G), 1))
    oh = oh.astype(jnp.float32)
    sums[...] += lax.dot_general(
        oh, h2, dimension_numbers=(((0,), (0,)), ((), ())),
        preferred_element_type=jnp.float32)
    counts[...] += jnp.broadcast_to(jnp.sum(oh, axis=0)[:, None], (G, H))

    @pl.when(i == NBLK - 1)
    def _():
        pooled = sums[...] / jnp.maximum(counts[...], 1.0)
        y = jnp.maximum(
            jnp.dot(pooled, l1w_ref[...], preferred_element_type=jnp.float32)
            + l1b_ref[...], 0.0)
        y = jnp.maximum(
            jnp.dot(y, l2w_ref[...], preferred_element_type=jnp.float32)
            + l2b_ref[...], 0.0)
        yv = jnp.sum(y * outw_ref[...], axis=1)[:, None] + outb_ref[...]
        o_ref[...] = jnp.broadcast_to(yv, (G, H))


def _mlp2(h, agg, batch3, w1, b1, w2, b2, l1w, l1b, l2w, l2b, outw, outb):
    return pl.pallas_call(
        _mlp2_body,
        grid=(NBLK,),
        in_specs=[
            pl.BlockSpec((BLK, F), lambda i: (i, 0)),
            pl.BlockSpec((2, BLK, F), lambda i: (0, i, 0)),
            pl.BlockSpec((1, 1, BLK), lambda i: (i, 0, 0)),
            pl.BlockSpec((H, H), lambda i: (0, 0)),
            pl.BlockSpec((1, H), lambda i: (0, 0)),
            pl.BlockSpec((H, H), lambda i: (0, 0)),
            pl.BlockSpec((1, H), lambda i: (0, 0)),
            pl.BlockSpec((H, H), lambda i: (0, 0)),
            pl.BlockSpec((1, H), lambda i: (0, 0)),
            pl.BlockSpec((H, H), lambda i: (0, 0)),
            pl.BlockSpec((1, H), lambda i: (0, 0)),
            pl.BlockSpec((1, H), lambda i: (0, 0)),
            pl.BlockSpec((1, 1), lambda i: (0, 0)),
        ],
        out_specs=pl.BlockSpec((G, H), lambda i: (0, 0)),
        out_shape=jax.ShapeDtypeStruct((G, H), jnp.float32),
        scratch_shapes=[
            pltpu.VMEM((G, H), jnp.float32),
            pltpu.VMEM((G, H), jnp.float32),
        ],
        compiler_params=pltpu.CompilerParams(
            dimension_semantics=("arbitrary",)),
    )(h, agg, batch3, w1, b1, w2, b2, l1w, l1b, l2w, l2b, outw, outb)


def kernel(x, edge_index, batch, g1_W1, g1_b1, g1_W2, g1_b2,
           g2_W1, g2_b1, g2_W2, g2_b2, l1_W, l1_b, l2_W, l2_b,
           out_W, out_b):
    src3 = edge_index[0].reshape(NW, NSTAGE, STAGE, K)
    dst3 = edge_index[1].reshape(NW, NSTAGE, STAGE, K)
    batch3 = batch.reshape(NBLK, 1, BLK)

    a1 = _sc_agg(x, src3, dst3)
    h1 = _mlp1(x, a1, g1_W1, g1_b1.reshape(1, H), g1_W2, g1_b2.reshape(1, H))
    a2 = _sc_agg(h1, src3, dst3)
    yf = _mlp2(h1, a2, batch3, g2_W1, g2_b1.reshape(1, H), g2_W2,
               g2_b2.reshape(1, H), l1_W, l1_b.reshape(1, H), l2_W,
               l2_b.reshape(1, H), out_W.reshape(1, H), out_b.reshape(1, 1))
    return yf[:, 0]
